# baseline (speedup 1.0000x reference)
import functools

import jax
import jax.numpy as jnp
from jax import lax
from jax.experimental import pallas as pl
from jax.experimental.pallas import tpu as pltpu
from jax.experimental.pallas import tpu_sc as plsc

NHID = 128
NLAYER = 5
N_NODES = 10000
N_EDGES = 320000
N_GRAPHS = 128

ATOM_DIMS = [119, 4, 12, 12, 10, 6, 6, 2, 2, 60]
BOND_DIMS = [5, 6, 2, 22]
ATOM_ROWS = [d + 2 for d in ATOM_DIMS]
BOND_ROWS = [d + 1 for d in BOND_DIMS]
ATOM_OFF = [sum(ATOM_ROWS[:i]) for i in range(len(ATOM_ROWS))]
BOND_OFF = [sum(BOND_ROWS[:i]) for i in range(len(BOND_ROWS))]
ATOM_TOT = sum(ATOM_ROWS)
BOND_TOT = sum(BOND_ROWS)
ATOM_PAD = 256
BOND_PAD = 40

SC_CORES = 2
SC_SUBCORES = 16
NW = SC_CORES * SC_SUBCORES
EDGES_PER_W = N_EDGES // NW
CHUNK = 64
NFULL = EDGES_PER_W // CHUNK
TAIL = EDGES_PER_W - NFULL * CHUNK
ROWS_PER_SUB = (N_NODES // SC_SUBCORES) // 8 * 8
ROWS_TAIL = N_NODES - ROWS_PER_SUB * SC_SUBCORES

NODE_BLK = 2000
NODE_GRID = N_NODES // NODE_BLK
EDGE_BLK = 4000
EDGE_GRID = N_EDGES // EDGE_BLK


def _atom_encoder_body(x_ref, tab_ref, out_ref):
    xb = x_ref[...]
    col = lax.broadcasted_iota(jnp.int32, (NODE_BLK, ATOM_PAD), 1)
    oh = jnp.zeros((NODE_BLK, ATOM_PAD), jnp.float32)
    for i, off in enumerate(ATOM_OFF):
        oh = oh + (col == xb[:, i][:, None] + off).astype(jnp.float32)
    h = jax.lax.dot_general(
        oh, tab_ref[...], (((1,), (0,)), ((), ())),
        preferred_element_type=jnp.float32)
    out_ref[...] = h


def _bond_encoder_body(a_ref, tab_ref, outpk_ref):
    ab = a_ref[...]
    col = lax.broadcasted_iota(jnp.int32, (EDGE_BLK, BOND_PAD), 1)
    oh = jnp.zeros((EDGE_BLK, BOND_PAD), jnp.float32)
    for i, off in enumerate(BOND_OFF):
        oh = oh + (col == ab[:, i][:, None] + off).astype(jnp.float32)
    e = jax.lax.dot_general(
        oh, tab_ref[...], (((1,), (0,)), ((), ())),
        preferred_element_type=jnp.float32)
    er = jnp.reshape(e.astype(jnp.bfloat16).astype(jnp.float32),
                     (EDGE_BLK // 2, 2, NHID))
    b_even = lax.bitcast_convert_type(er[:, 0, :], jnp.int32)
    b_odd = lax.bitcast_convert_type(er[:, 1, :], jnp.int32)
    outpk_ref[...] = jnp.bitwise_or(
        lax.shift_right_logical(b_even, 16), b_odd)


def _atom_encoder(x, atom_tab):
    return pl.pallas_call(
        _atom_encoder_body,
        grid=(NODE_GRID,),
        in_specs=[
            pl.BlockSpec((NODE_BLK, len(ATOM_DIMS)), lambda i: (i, 0)),
            pl.BlockSpec((ATOM_PAD, NHID), lambda i: (0, 0)),
        ],
        out_specs=pl.BlockSpec((NODE_BLK, NHID), lambda i: (i, 0)),
        out_shape=jax.ShapeDtypeStruct((N_NODES, NHID), jnp.float32),
    )(x, atom_tab)


def _bond_encoder(edge_attr, bond_tab):
    return pl.pallas_call(
        _bond_encoder_body,
        grid=(EDGE_GRID,),
        in_specs=[
            pl.BlockSpec((EDGE_BLK, len(BOND_DIMS)), lambda i: (i, 0)),
            pl.BlockSpec((BOND_PAD, NHID), lambda i: (0, 0)),
        ],
        out_specs=pl.BlockSpec((EDGE_BLK // 2, NHID), lambda i: (i, 0)),
        out_shape=jax.ShapeDtypeStruct((N_EDGES // 2, NHID), jnp.int32),
    )(edge_attr, bond_tab)


def _decode_pair(w):
    lo = lax.bitcast_convert_type(w * 65536, jnp.float32)
    hi = lax.bitcast_convert_type(jnp.bitwise_and(w, -65536), jnp.float32)
    return lo, hi


def _relu_add_rows(msg_v, eview, p):
    i0 = 2 * p
    for j in range(NHID // 16):
        sl = pl.ds(j * 16, 16)
        ew = eview[p, sl]
        e_lo, e_hi = _decode_pair(ew)
        msg_v[i0, sl] = jnp.maximum(msg_v[i0, sl] + e_lo, 0.0)
        msg_v[i0 + 1, sl] = jnp.maximum(msg_v[i0 + 1, sl] + e_hi, 0.0)


@functools.partial(
    pl.kernel,
    out_type=jax.ShapeDtypeStruct((SC_CORES, N_NODES, NHID), jnp.float32),
    mesh=plsc.VectorSubcoreMesh(core_axis_name="c", subcore_axis_name="s"),
    scratch_types=[
        pltpu.VMEM((2, 2, CHUNK), jnp.int32),
        pltpu.VMEM((TAIL,), jnp.int32),
        pltpu.VMEM((TAIL,), jnp.int32),
        pltpu.VMEM((CHUNK, NHID), jnp.float32),
        pltpu.VMEM((CHUNK, NHID), jnp.float32),
        pltpu.VMEM((CHUNK // 2, NHID), jnp.int32),
        pltpu.VMEM((CHUNK // 2, NHID), jnp.int32),
        pltpu.VMEM_SHARED((N_NODES, NHID), jnp.float32),
        pltpu.SemaphoreType.DMA,
        pltpu.SemaphoreType.DMA,
        pltpu.SemaphoreType.DMA,
        pltpu.SemaphoreType.DMA,
        pltpu.SemaphoreType.DMA,
        pltpu.SemaphoreType.DMA,
        pltpu.SemaphoreType.DMA,
        pltpu.SemaphoreType.DMA,
    ],
)
def _edge_agg(h_hbm, e_hbm, src_hbm, dst_hbm, zeros_hbm, out_hbm,
              idx_v, srct_v, dstt_v, msg0, msg1, e0, e1, agg_sh,
              si0, si1, se0, se1, sg0, sg1, ss0, ss1):
    cid = lax.axis_index("c")
    sid = lax.axis_index("s")
    wid = sid * SC_CORES + cid
    stripe = pl.ds(sid * ROWS_PER_SUB, ROWS_PER_SUB)
    tail_stripe = pl.ds(SC_SUBCORES * ROWS_PER_SUB, ROWS_TAIL)

    msg = (msg0, msg1)
    ebuf = (e0, e1)
    sem_i = (si0, si1)
    sem_e = (se0, se1)
    sem_g = (sg0, sg1)
    sem_s = (ss0, ss1)

    def kernel_body():
        pltpu.sync_copy(zeros_hbm.at[stripe], agg_sh.at[stripe])

        @pl.when(sid == 0)
        def _():
            pltpu.sync_copy(zeros_hbm.at[tail_stripe], agg_sh.at[tail_stripe])

        plsc.subcore_barrier()

        base0 = wid * EDGES_PER_W

        def start_loads(g, b):
            base = base0 + g * CHUNK
            pbase = wid * (EDGES_PER_W // 2) + g * (CHUNK // 2)
            pltpu.async_copy(src_hbm.at[pl.ds(base, CHUNK)],
                             idx_v.at[b, 0], sem_i[b])
            pltpu.async_copy(dst_hbm.at[pl.ds(base, CHUNK)],
                             idx_v.at[b, 1], sem_i[b])
            pltpu.async_copy(e_hbm.at[pl.ds(pbase, CHUNK // 2)], ebuf[b],
                             sem_e[b])

        def wait_idx(g, b):
            base = base0 + g * CHUNK
            pltpu.make_async_copy(src_hbm.at[pl.ds(base, CHUNK)],
                                  idx_v.at[b, 0], sem_i[b]).wait()
            pltpu.make_async_copy(dst_hbm.at[pl.ds(base, CHUNK)],
                                  idx_v.at[b, 1], sem_i[b]).wait()

        def wait_e(b):
            pltpu.make_async_copy(e_hbm.at[pl.ds(0, CHUNK // 2)],
                                  ebuf[b], sem_e[b]).wait()

        def start_gather(b):
            pltpu.async_copy(h_hbm.at[idx_v.at[b, 0]], msg[b], sem_g[b])

        def wait_gather(b):
            pltpu.make_async_copy(h_hbm.at[idx_v.at[b, 0]],
                                  msg[b], sem_g[b]).wait()

        def start_scatter(b):
            pltpu.async_copy(msg[b], agg_sh.at[idx_v.at[b, 1]], sem_s[b],
                             add=True)

        def wait_scatter(b):
            pltpu.make_async_copy(msg[b], agg_sh.at[idx_v.at[b, 1]],
                                  sem_s[b]).wait()

        def compute(b):
            def row_body(p, c):
                _relu_add_rows(msg[b], ebuf[b], p)
                return c

            lax.fori_loop(0, CHUNK // 2, row_body, 0)

        start_loads(0, 0)
        start_loads(1, 1)
        wait_idx(0, 0)
        start_gather(0)

        def pair_body(kk, carry):
            g = 2 * kk
            wait_idx(g + 1, 1)

            @pl.when(kk > 0)
            def _():
                wait_scatter(1)

            start_gather(1)
            wait_e(0)
            wait_gather(0)
            compute(0)
            start_scatter(0)

            @pl.when(kk < (NFULL // 2) - 1)
            def _():
                start_loads(g + 2, 0)

            @pl.when(kk < (NFULL // 2) - 1)
            def _():
                wait_idx(g + 2, 0)
                wait_scatter(0)
                start_gather(0)

            wait_e(1)
            wait_gather(1)
            compute(1)
            start_scatter(1)

            @pl.when(kk < (NFULL // 2) - 1)
            def _():
                start_loads(g + 3, 1)

            return carry

        lax.fori_loop(0, NFULL // 2, pair_body, 0)
        wait_scatter(0)
        wait_scatter(1)

        if TAIL:
            base = base0 + NFULL * CHUNK
            pltpu.sync_copy(src_hbm.at[pl.ds(base, TAIL)], srct_v)
            pltpu.sync_copy(dst_hbm.at[pl.ds(base, TAIL)], dstt_v)
            pbase = wid * (EDGES_PER_W // 2) + NFULL * (CHUNK // 2)
            pltpu.sync_copy(e_hbm.at[pl.ds(pbase, TAIL // 2)],
                            e0.at[pl.ds(0, TAIL // 2)])
            pltpu.async_copy(h_hbm.at[srct_v], msg0.at[pl.ds(0, TAIL)],
                             sg0).wait()

            def trow_body(p, c):
                _relu_add_rows(msg0, e0, p)
                return c

            lax.fori_loop(0, TAIL // 2, trow_body, 0)
            pltpu.sync_copy(msg0.at[pl.ds(0, TAIL)], agg_sh.at[dstt_v],
                            add=True)

        plsc.subcore_barrier()
        pltpu.sync_copy(agg_sh.at[stripe], out_hbm.at[cid, stripe])

        @pl.when(sid == 0)
        def _():
            pltpu.sync_copy(agg_sh.at[tail_stripe],
                            out_hbm.at[cid, tail_stripe])

    kernel_body()


def _mlp_body(eps_ref, h_ref, agg_ref, w1_ref, b1_ref, w2_ref, b2_ref,
              batch_ref, hout_ref, pool_ref):
    eps = eps_ref[0, 0]
    z = h_ref[...] * (1.0 + eps) + agg_ref[0] + agg_ref[1]
    t = jnp.maximum(
        jax.lax.dot_general(z, w1_ref[...], (((1,), (0,)), ((), ())),
                            preferred_element_type=jnp.float32)
        + b1_ref[0][None, :], 0.0)
    o = jnp.maximum(
        jax.lax.dot_general(t, w2_ref[...], (((1,), (0,)), ((), ())),
                            preferred_element_type=jnp.float32)
        + b2_ref[0][None, :], 0.0)
    hout_ref[...] = o
    b = batch_ref[0, 0, :]
    oh = (b[:, None] == lax.broadcasted_iota(jnp.int32, (NODE_BLK, N_GRAPHS), 1)
          ).astype(jnp.float32)
    contrib = jax.lax.dot_general(oh, o, (((0,), (0,)), ((), ())),
                                  preferred_element_type=jnp.float32)

    @pl.when(pl.program_id(0) == 0)
    def _():
        pool_ref[...] = contrib

    @pl.when(pl.program_id(0) != 0)
    def _():
        pool_ref[...] = pool_ref[...] + contrib


def _mlp_layer(eps, h, agg2, w1, b1, w2, b2, batch3):
    return pl.pallas_call(
        _mlp_body,
        grid=(NODE_GRID,),
        in_specs=[
            pl.BlockSpec(memory_space=pltpu.SMEM),
            pl.BlockSpec((NODE_BLK, NHID), lambda i: (i, 0)),
            pl.BlockSpec((SC_CORES, NODE_BLK, NHID), lambda i: (0, i, 0)),
            pl.BlockSpec((NHID, 2 * NHID), lambda i: (0, 0)),
            pl.BlockSpec((1, 2 * NHID), lambda i: (0, 0)),
            pl.BlockSpec((2 * NHID, NHID), lambda i: (0, 0)),
            pl.BlockSpec((1, NHID), lambda i: (0, 0)),
            pl.BlockSpec((1, 1, NODE_BLK), lambda i: (i, 0, 0)),
        ],
        out_specs=[
            pl.BlockSpec((NODE_BLK, NHID), lambda i: (i, 0)),
            pl.BlockSpec((N_GRAPHS, NHID), lambda i: (0, 0)),
        ],
        out_shape=[
            jax.ShapeDtypeStruct((N_NODES, NHID), jnp.float32),
            jax.ShapeDtypeStruct((N_GRAPHS, NHID), jnp.float32),
        ],
    )(eps, h, agg2, w1, b1, w2, b2, batch3)


def kernel(x, edge_index, edge_attr, batch, atom_tables, bond_tables,
           mlp_params):
    atom_tab = jnp.zeros((ATOM_PAD, NHID), jnp.float32)
    atom_tab = atom_tab.at[:ATOM_TOT].set(jnp.concatenate(atom_tables, axis=0))
    bond_tab = jnp.zeros((BOND_PAD, NHID), jnp.float32)
    bond_tab = bond_tab.at[:BOND_TOT].set(jnp.concatenate(bond_tables, axis=0))

    h = _atom_encoder(x, atom_tab)
    e16 = _bond_encoder(edge_attr, bond_tab)

    src = edge_index[0]
    dst = edge_index[1]
    zeros = jnp.zeros((N_NODES, NHID), jnp.float32)
    batch3 = batch.reshape(NODE_GRID, 1, NODE_BLK)

    hs = []
    pools = []
    for (w1, b1, w2, b2, eps) in mlp_params:
        agg2 = _edge_agg(h, e16, src, dst, zeros)
        h, pool = _mlp_layer(
            jnp.reshape(eps, (1, 1)), h, agg2, w1,
            jnp.reshape(b1, (1, 2 * NHID)), w2, jnp.reshape(b2, (1, NHID)),
            batch3)
        hs.append(h)
        pools.append(pool)

    node_embs = jnp.concatenate(hs, axis=-1)
    graph_embs = jnp.concatenate(pools, axis=-1)
    return (graph_embs, node_embs)


# speedup vs baseline: 1.0004x; 1.0004x over previous
import functools

import jax
import jax.numpy as jnp
from jax import lax
from jax.experimental import pallas as pl
from jax.experimental.pallas import tpu as pltpu
from jax.experimental.pallas import tpu_sc as plsc

NHID = 128
NLAYER = 5
N_NODES = 10000
N_EDGES = 320000
N_GRAPHS = 128

ATOM_DIMS = [119, 4, 12, 12, 10, 6, 6, 2, 2, 60]
BOND_DIMS = [5, 6, 2, 22]
ATOM_ROWS = [d + 2 for d in ATOM_DIMS]
BOND_ROWS = [d + 1 for d in BOND_DIMS]
ATOM_OFF = [sum(ATOM_ROWS[:i]) for i in range(len(ATOM_ROWS))]
BOND_OFF = [sum(BOND_ROWS[:i]) for i in range(len(BOND_ROWS))]
ATOM_TOT = sum(ATOM_ROWS)
BOND_TOT = sum(BOND_ROWS)
ATOM_PAD = 256
BOND_PAD = 40

SC_CORES = 2
SC_SUBCORES = 16
NW = SC_CORES * SC_SUBCORES
EDGES_PER_W = N_EDGES // NW
CHUNK = 64
NFULL = EDGES_PER_W // CHUNK
TAIL = EDGES_PER_W - NFULL * CHUNK
ROWS_PER_SUB = (N_NODES // SC_SUBCORES) // 8 * 8
ROWS_TAIL = N_NODES - ROWS_PER_SUB * SC_SUBCORES

NODE_BLK = 2000
NODE_GRID = N_NODES // NODE_BLK
EDGE_BLK = 4000
EDGE_GRID = N_EDGES // EDGE_BLK


def _atom_encoder_body(x_ref, tab_ref, out_ref):
    xb = x_ref[...]
    col = lax.broadcasted_iota(jnp.int32, (NODE_BLK, ATOM_PAD), 1)
    oh = jnp.zeros((NODE_BLK, ATOM_PAD), jnp.float32)
    for i, off in enumerate(ATOM_OFF):
        oh = oh + (col == xb[:, i][:, None] + off).astype(jnp.float32)
    h = jax.lax.dot_general(
        oh, tab_ref[...], (((1,), (0,)), ((), ())),
        preferred_element_type=jnp.float32)
    out_ref[...] = h


def _bond_encoder_body(a_ref, tab_ref, outpk_ref):
    ab = a_ref[...]
    col = lax.broadcasted_iota(jnp.int32, (EDGE_BLK, BOND_PAD), 1)
    oh = jnp.zeros((EDGE_BLK, BOND_PAD), jnp.float32)
    for i, off in enumerate(BOND_OFF):
        oh = oh + (col == ab[:, i][:, None] + off).astype(jnp.float32)
    e = jax.lax.dot_general(
        oh, tab_ref[...], (((1,), (0,)), ((), ())),
        preferred_element_type=jnp.float32)
    er = jnp.reshape(e.astype(jnp.bfloat16).astype(jnp.float32),
                     (EDGE_BLK // 2, 2, NHID))
    b_even = lax.bitcast_convert_type(er[:, 0, :], jnp.int32)
    b_odd = lax.bitcast_convert_type(er[:, 1, :], jnp.int32)
    outpk_ref[...] = jnp.bitwise_or(
        lax.shift_right_logical(b_even, 16), b_odd)


def _atom_encoder(x, atom_tab):
    return pl.pallas_call(
        _atom_encoder_body,
        grid=(NODE_GRID,),
        in_specs=[
            pl.BlockSpec((NODE_BLK, len(ATOM_DIMS)), lambda i: (i, 0)),
            pl.BlockSpec((ATOM_PAD, NHID), lambda i: (0, 0)),
        ],
        out_specs=pl.BlockSpec((NODE_BLK, NHID), lambda i: (i, 0)),
        out_shape=jax.ShapeDtypeStruct((N_NODES, NHID), jnp.float32),
    )(x, atom_tab)


def _bond_encoder(edge_attr, bond_tab):
    return pl.pallas_call(
        _bond_encoder_body,
        grid=(EDGE_GRID,),
        in_specs=[
            pl.BlockSpec((EDGE_BLK, len(BOND_DIMS)), lambda i: (i, 0)),
            pl.BlockSpec((BOND_PAD, NHID), lambda i: (0, 0)),
        ],
        out_specs=pl.BlockSpec((EDGE_BLK // 2, NHID), lambda i: (i, 0)),
        out_shape=jax.ShapeDtypeStruct((N_EDGES // 2, NHID), jnp.int32),
    )(edge_attr, bond_tab)


def _decode_pair(w):
    lo = lax.bitcast_convert_type(lax.shift_left(w, 16), jnp.float32)
    hi = lax.bitcast_convert_type(jnp.bitwise_and(w, -65536), jnp.float32)
    return lo, hi


def _relu_add_rows(msg_v, eview, p):
    i0 = 2 * p
    for j in range(NHID // 16):
        sl = pl.ds(j * 16, 16)
        ew = eview[p, sl]
        e_lo, e_hi = _decode_pair(ew)
        msg_v[i0, sl] = jnp.maximum(msg_v[i0, sl] + e_lo, 0.0)
        msg_v[i0 + 1, sl] = jnp.maximum(msg_v[i0 + 1, sl] + e_hi, 0.0)


@functools.partial(
    pl.kernel,
    out_type=jax.ShapeDtypeStruct((SC_CORES, N_NODES, NHID), jnp.float32),
    mesh=plsc.VectorSubcoreMesh(core_axis_name="c", subcore_axis_name="s"),
    scratch_types=[
        pltpu.VMEM((2, 2, CHUNK), jnp.int32),
        pltpu.VMEM((TAIL,), jnp.int32),
        pltpu.VMEM((TAIL,), jnp.int32),
        pltpu.VMEM((CHUNK, NHID), jnp.float32),
        pltpu.VMEM((CHUNK, NHID), jnp.float32),
        pltpu.VMEM((CHUNK // 2, NHID), jnp.int32),
        pltpu.VMEM((CHUNK // 2, NHID), jnp.int32),
        pltpu.VMEM_SHARED((N_NODES, NHID), jnp.float32),
        pltpu.SemaphoreType.DMA,
        pltpu.SemaphoreType.DMA,
        pltpu.SemaphoreType.DMA,
        pltpu.SemaphoreType.DMA,
        pltpu.SemaphoreType.DMA,
        pltpu.SemaphoreType.DMA,
        pltpu.SemaphoreType.DMA,
        pltpu.SemaphoreType.DMA,
    ],
)
def _edge_agg(h_hbm, e_hbm, src_hbm, dst_hbm, zeros_hbm, out_hbm,
              idx_v, srct_v, dstt_v, msg0, msg1, e0, e1, agg_sh,
              si0, si1, se0, se1, sg0, sg1, ss0, ss1):
    cid = lax.axis_index("c")
    sid = lax.axis_index("s")
    wid = sid * SC_CORES + cid
    stripe = pl.ds(sid * ROWS_PER_SUB, ROWS_PER_SUB)
    tail_stripe = pl.ds(SC_SUBCORES * ROWS_PER_SUB, ROWS_TAIL)

    msg = (msg0, msg1)
    ebuf = (e0, e1)
    sem_i = (si0, si1)
    sem_e = (se0, se1)
    sem_g = (sg0, sg1)
    sem_s = (ss0, ss1)

    def kernel_body():
        pltpu.sync_copy(zeros_hbm.at[stripe], agg_sh.at[stripe])

        @pl.when(sid == 0)
        def _():
            pltpu.sync_copy(zeros_hbm.at[tail_stripe], agg_sh.at[tail_stripe])

        plsc.subcore_barrier()

        base0 = wid * EDGES_PER_W

        def start_loads(g, b):
            base = base0 + g * CHUNK
            pbase = wid * (EDGES_PER_W // 2) + g * (CHUNK // 2)
            pltpu.async_copy(src_hbm.at[pl.ds(base, CHUNK)],
                             idx_v.at[b, 0], sem_i[b])
            pltpu.async_copy(dst_hbm.at[pl.ds(base, CHUNK)],
                             idx_v.at[b, 1], sem_i[b])
            pltpu.async_copy(e_hbm.at[pl.ds(pbase, CHUNK // 2)], ebuf[b],
                             sem_e[b])

        def wait_idx(g, b):
            base = base0 + g * CHUNK
            pltpu.make_async_copy(src_hbm.at[pl.ds(base, CHUNK)],
                                  idx_v.at[b, 0], sem_i[b]).wait()
            pltpu.make_async_copy(dst_hbm.at[pl.ds(base, CHUNK)],
                                  idx_v.at[b, 1], sem_i[b]).wait()

        def wait_e(b):
            pltpu.make_async_copy(e_hbm.at[pl.ds(0, CHUNK // 2)],
                                  ebuf[b], sem_e[b]).wait()

        def start_gather(b):
            pltpu.async_copy(h_hbm.at[idx_v.at[b, 0]], msg[b], sem_g[b])

        def wait_gather(b):
            pltpu.make_async_copy(h_hbm.at[idx_v.at[b, 0]],
                                  msg[b], sem_g[b]).wait()

        def start_scatter(b):
            pltpu.async_copy(msg[b], agg_sh.at[idx_v.at[b, 1]], sem_s[b],
                             add=True)

        def wait_scatter(b):
            pltpu.make_async_copy(msg[b], agg_sh.at[idx_v.at[b, 1]],
                                  sem_s[b]).wait()

        def compute(b):
            def row_body(p, c):
                _relu_add_rows(msg[b], ebuf[b], p)
                return c

            lax.fori_loop(0, CHUNK // 2, row_body, 0)

        start_loads(0, 0)
        start_loads(1, 1)
        wait_idx(0, 0)
        start_gather(0)

        def pair_body(kk, carry):
            g = 2 * kk
            wait_idx(g + 1, 1)

            @pl.when(kk > 0)
            def _():
                wait_scatter(1)

            start_gather(1)
            wait_e(0)
            wait_gather(0)
            compute(0)
            start_scatter(0)

            @pl.when(kk < (NFULL // 2) - 1)
            def _():
                start_loads(g + 2, 0)

            @pl.when(kk < (NFULL // 2) - 1)
            def _():
                wait_idx(g + 2, 0)
                wait_scatter(0)
                start_gather(0)

            wait_e(1)
            wait_gather(1)
            compute(1)
            start_scatter(1)

            @pl.when(kk < (NFULL // 2) - 1)
            def _():
                start_loads(g + 3, 1)

            return carry

        lax.fori_loop(0, NFULL // 2, pair_body, 0)
        wait_scatter(0)
        wait_scatter(1)

        if TAIL:
            base = base0 + NFULL * CHUNK
            pltpu.sync_copy(src_hbm.at[pl.ds(base, TAIL)], srct_v)
            pltpu.sync_copy(dst_hbm.at[pl.ds(base, TAIL)], dstt_v)
            pbase = wid * (EDGES_PER_W // 2) + NFULL * (CHUNK // 2)
            pltpu.sync_copy(e_hbm.at[pl.ds(pbase, TAIL // 2)],
                            e0.at[pl.ds(0, TAIL // 2)])
            pltpu.async_copy(h_hbm.at[srct_v], msg0.at[pl.ds(0, TAIL)],
                             sg0).wait()

            def trow_body(p, c):
                _relu_add_rows(msg0, e0, p)
                return c

            lax.fori_loop(0, TAIL // 2, trow_body, 0)
            pltpu.sync_copy(msg0.at[pl.ds(0, TAIL)], agg_sh.at[dstt_v],
                            add=True)

        plsc.subcore_barrier()
        pltpu.sync_copy(agg_sh.at[stripe], out_hbm.at[cid, stripe])

        @pl.when(sid == 0)
        def _():
            pltpu.sync_copy(agg_sh.at[tail_stripe],
                            out_hbm.at[cid, tail_stripe])

    kernel_body()


def _mlp_body(eps_ref, h_ref, agg_ref, w1_ref, b1_ref, w2_ref, b2_ref,
              batch_ref, hout_ref, pool_ref):
    eps = eps_ref[0, 0]
    z = h_ref[...] * (1.0 + eps) + agg_ref[0] + agg_ref[1]
    t = jnp.maximum(
        jax.lax.dot_general(z, w1_ref[...], (((1,), (0,)), ((), ())),
                            preferred_element_type=jnp.float32)
        + b1_ref[0][None, :], 0.0)
    o = jnp.maximum(
        jax.lax.dot_general(t, w2_ref[...], (((1,), (0,)), ((), ())),
                            preferred_element_type=jnp.float32)
        + b2_ref[0][None, :], 0.0)
    hout_ref[...] = o
    b = batch_ref[0, 0, :]
    oh = (b[:, None] == lax.broadcasted_iota(jnp.int32, (NODE_BLK, N_GRAPHS), 1)
          ).astype(jnp.float32)
    contrib = jax.lax.dot_general(oh, o, (((0,), (0,)), ((), ())),
                                  preferred_element_type=jnp.float32)

    @pl.when(pl.program_id(0) == 0)
    def _():
        pool_ref[...] = contrib

    @pl.when(pl.program_id(0) != 0)
    def _():
        pool_ref[...] = pool_ref[...] + contrib


def _mlp_layer(eps, h, agg2, w1, b1, w2, b2, batch3):
    return pl.pallas_call(
        _mlp_body,
        grid=(NODE_GRID,),
        in_specs=[
            pl.BlockSpec(memory_space=pltpu.SMEM),
            pl.BlockSpec((NODE_BLK, NHID), lambda i: (i, 0)),
            pl.BlockSpec((SC_CORES, NODE_BLK, NHID), lambda i: (0, i, 0)),
            pl.BlockSpec((NHID, 2 * NHID), lambda i: (0, 0)),
            pl.BlockSpec((1, 2 * NHID), lambda i: (0, 0)),
            pl.BlockSpec((2 * NHID, NHID), lambda i: (0, 0)),
            pl.BlockSpec((1, NHID), lambda i: (0, 0)),
            pl.BlockSpec((1, 1, NODE_BLK), lambda i: (i, 0, 0)),
        ],
        out_specs=[
            pl.BlockSpec((NODE_BLK, NHID), lambda i: (i, 0)),
            pl.BlockSpec((N_GRAPHS, NHID), lambda i: (0, 0)),
        ],
        out_shape=[
            jax.ShapeDtypeStruct((N_NODES, NHID), jnp.float32),
            jax.ShapeDtypeStruct((N_GRAPHS, NHID), jnp.float32),
        ],
    )(eps, h, agg2, w1, b1, w2, b2, batch3)


def kernel(x, edge_index, edge_attr, batch, atom_tables, bond_tables,
           mlp_params):
    atom_tab = jnp.zeros((ATOM_PAD, NHID), jnp.float32)
    atom_tab = atom_tab.at[:ATOM_TOT].set(jnp.concatenate(atom_tables, axis=0))
    bond_tab = jnp.zeros((BOND_PAD, NHID), jnp.float32)
    bond_tab = bond_tab.at[:BOND_TOT].set(jnp.concatenate(bond_tables, axis=0))

    h = _atom_encoder(x, atom_tab)
    e16 = _bond_encoder(edge_attr, bond_tab)

    src = edge_index[0]
    dst = edge_index[1]
    zeros = jnp.zeros((N_NODES, NHID), jnp.float32)
    batch3 = batch.reshape(NODE_GRID, 1, NODE_BLK)

    hs = []
    pools = []
    for (w1, b1, w2, b2, eps) in mlp_params:
        agg2 = _edge_agg(h, e16, src, dst, zeros)
        h, pool = _mlp_layer(
            jnp.reshape(eps, (1, 1)), h, agg2, w1,
            jnp.reshape(b1, (1, 2 * NHID)), w2, jnp.reshape(b2, (1, NHID)),
            batch3)
        hs.append(h)
        pools.append(pool)

    node_embs = jnp.concatenate(hs, axis=-1)
    graph_embs = jnp.concatenate(pools, axis=-1)
    return (graph_embs, node_embs)


# speedup vs baseline: 1.6057x; 1.6051x over previous
import functools

import jax
import jax.numpy as jnp
from jax import lax
from jax.experimental import pallas as pl
from jax.experimental.pallas import tpu as pltpu
from jax.experimental.pallas import tpu_sc as plsc

NHID = 128
NLAYER = 5
N_NODES = 10000
N_EDGES = 320000
N_GRAPHS = 128

ATOM_DIMS = [119, 4, 12, 12, 10, 6, 6, 2, 2, 60]
BOND_DIMS = [5, 6, 2, 22]
ATOM_ROWS = [d + 2 for d in ATOM_DIMS]
BOND_ROWS = [d + 1 for d in BOND_DIMS]
ATOM_OFF = [sum(ATOM_ROWS[:i]) for i in range(len(ATOM_ROWS))]
BOND_OFF = [sum(BOND_ROWS[:i]) for i in range(len(BOND_ROWS))]
ATOM_TOT = sum(ATOM_ROWS)
BOND_TOT = sum(BOND_ROWS)
ATOM_PAD = 256
BOND_PAD = 40

SC_CORES = 2
SC_SUBCORES = 16
NW = SC_CORES * SC_SUBCORES
EDGES_PER_W = N_EDGES // NW
CHUNK = 64
NFULL = EDGES_PER_W // CHUNK
TAIL = EDGES_PER_W - NFULL * CHUNK
ROWS_PER_SUB = (N_NODES // SC_SUBCORES) // 8 * 8
ROWS_TAIL = N_NODES - ROWS_PER_SUB * SC_SUBCORES

NODE_BLK = 2000
NODE_GRID = N_NODES // NODE_BLK
EDGE_BLK = 4000
EDGE_GRID = N_EDGES // EDGE_BLK


def _atom_encoder_body(x_ref, tab_ref, out_ref):
    xb = x_ref[...]
    col = lax.broadcasted_iota(jnp.int32, (NODE_BLK, ATOM_PAD), 1)
    oh = jnp.zeros((NODE_BLK, ATOM_PAD), jnp.float32)
    for i, off in enumerate(ATOM_OFF):
        oh = oh + (col == xb[:, i][:, None] + off).astype(jnp.float32)
    h = jax.lax.dot_general(
        oh, tab_ref[...], (((1,), (0,)), ((), ())),
        preferred_element_type=jnp.float32)
    out_ref[...] = h


def _bond_encoder_body(a_ref, tab_ref, outpk_ref):
    ab = a_ref[...]
    col = lax.broadcasted_iota(jnp.int32, (EDGE_BLK, BOND_PAD), 1)
    oh = jnp.zeros((EDGE_BLK, BOND_PAD), jnp.float32)
    for i, off in enumerate(BOND_OFF):
        oh = oh + (col == ab[:, i][:, None] + off).astype(jnp.float32)
    e = jax.lax.dot_general(
        oh, tab_ref[...], (((1,), (0,)), ((), ())),
        preferred_element_type=jnp.float32)
    er = jnp.reshape(e.astype(jnp.bfloat16).astype(jnp.float32),
                     (EDGE_BLK // 2, 2, NHID))
    b_even = lax.bitcast_convert_type(er[:, 0, :], jnp.int32)
    b_odd = lax.bitcast_convert_type(er[:, 1, :], jnp.int32)
    outpk_ref[...] = jnp.bitwise_or(
        lax.shift_right_logical(b_even, 16), b_odd)


def _atom_encoder(x, atom_tab):
    return pl.pallas_call(
        _atom_encoder_body,
        grid=(NODE_GRID,),
        in_specs=[
            pl.BlockSpec((NODE_BLK, len(ATOM_DIMS)), lambda i: (i, 0)),
            pl.BlockSpec((ATOM_PAD, NHID), lambda i: (0, 0)),
        ],
        out_specs=pl.BlockSpec((NODE_BLK, NHID), lambda i: (i, 0)),
        out_shape=jax.ShapeDtypeStruct((N_NODES, NHID), jnp.float32),
    )(x, atom_tab)


def _bond_encoder(edge_attr, bond_tab):
    return pl.pallas_call(
        _bond_encoder_body,
        grid=(EDGE_GRID,),
        in_specs=[
            pl.BlockSpec((EDGE_BLK, len(BOND_DIMS)), lambda i: (i, 0)),
            pl.BlockSpec((BOND_PAD, NHID), lambda i: (0, 0)),
        ],
        out_specs=pl.BlockSpec((EDGE_BLK // 2, NHID), lambda i: (i, 0)),
        out_shape=jax.ShapeDtypeStruct((N_EDGES // 2, NHID), jnp.int32),
    )(edge_attr, bond_tab)


def _decode_pair(w):
    lo = lax.bitcast_convert_type(lax.shift_left(w, 16), jnp.float32)
    hi = lax.bitcast_convert_type(jnp.bitwise_and(w, -65536), jnp.float32)
    return lo, hi


def _relu_add_rows(msg_v, eview, p):
    i0 = 2 * p
    for j in range(NHID // 16):
        sl = pl.ds(j * 16, 16)
        ew = eview[p, sl]
        e_lo, e_hi = _decode_pair(ew)
        msg_v[i0, sl] = jnp.maximum(msg_v[i0, sl] + e_lo, 0.0)
        msg_v[i0 + 1, sl] = jnp.maximum(msg_v[i0 + 1, sl] + e_hi, 0.0)


@functools.partial(
    pl.kernel,
    out_type=jax.ShapeDtypeStruct((SC_CORES, N_NODES, NHID), jnp.float32),
    mesh=plsc.VectorSubcoreMesh(core_axis_name="c", subcore_axis_name="s"),
    scratch_types=[
        pltpu.VMEM((2, 2, CHUNK), jnp.int32),
        pltpu.VMEM((TAIL,), jnp.int32),
        pltpu.VMEM((TAIL,), jnp.int32),
        pltpu.VMEM((CHUNK, NHID), jnp.float32),
        pltpu.VMEM((CHUNK, NHID), jnp.float32),
        pltpu.VMEM((CHUNK // 2, NHID), jnp.int32),
        pltpu.VMEM((CHUNK // 2, NHID), jnp.int32),
        pltpu.VMEM_SHARED((N_NODES, NHID), jnp.float32),
        pltpu.SemaphoreType.DMA,
        pltpu.SemaphoreType.DMA,
        pltpu.SemaphoreType.DMA,
        pltpu.SemaphoreType.DMA,
        pltpu.SemaphoreType.DMA,
        pltpu.SemaphoreType.DMA,
        pltpu.SemaphoreType.DMA,
        pltpu.SemaphoreType.DMA,
    ],
)
def _edge_agg(h_hbm, e_hbm, src_hbm, dst_hbm, zeros_hbm, out_hbm,
              idx_v, srct_v, dstt_v, msg0, msg1, e0, e1, agg_sh,
              si0, si1, se0, se1, sg0, sg1, ss0, ss1):
    cid = lax.axis_index("c")
    sid = lax.axis_index("s")
    wid = sid * SC_CORES + cid
    stripe = pl.ds(sid * ROWS_PER_SUB, ROWS_PER_SUB)
    tail_stripe = pl.ds(SC_SUBCORES * ROWS_PER_SUB, ROWS_TAIL)

    msg = (msg0, msg1)
    ebuf = (e0, e1)
    sem_i = (si0, si1)
    sem_e = (se0, se1)
    sem_g = (sg0, sg1)
    sem_s = (ss0, ss1)

    def kernel_body():
        pltpu.sync_copy(zeros_hbm.at[stripe], agg_sh.at[stripe])

        @pl.when(sid == 0)
        def _():
            pltpu.sync_copy(zeros_hbm.at[tail_stripe], agg_sh.at[tail_stripe])

        plsc.subcore_barrier()

        base0 = wid * EDGES_PER_W

        def start_loads(g, b):
            base = base0 + g * CHUNK
            pbase = wid * (EDGES_PER_W // 2) + g * (CHUNK // 2)
            pltpu.async_copy(src_hbm.at[pl.ds(base, CHUNK)],
                             idx_v.at[b, 0], sem_i[b])
            pltpu.async_copy(dst_hbm.at[pl.ds(base, CHUNK)],
                             idx_v.at[b, 1], sem_i[b])
            pltpu.async_copy(e_hbm.at[pl.ds(pbase, CHUNK // 2)], ebuf[b],
                             sem_e[b])

        def wait_idx(g, b):
            base = base0 + g * CHUNK
            pltpu.make_async_copy(src_hbm.at[pl.ds(base, CHUNK)],
                                  idx_v.at[b, 0], sem_i[b]).wait()
            pltpu.make_async_copy(dst_hbm.at[pl.ds(base, CHUNK)],
                                  idx_v.at[b, 1], sem_i[b]).wait()

        def wait_e(b):
            pltpu.make_async_copy(e_hbm.at[pl.ds(0, CHUNK // 2)],
                                  ebuf[b], sem_e[b]).wait()

        def start_gather(b):
            pltpu.async_copy(h_hbm.at[idx_v.at[b, 0]], msg[b], sem_g[b])

        def wait_gather(b):
            pltpu.make_async_copy(h_hbm.at[idx_v.at[b, 0]],
                                  msg[b], sem_g[b]).wait()

        def start_scatter(b):
            pltpu.async_copy(msg[b], agg_sh.at[idx_v.at[b, 1]], sem_s[b],
                             add=True)

        def wait_scatter(b):
            pltpu.make_async_copy(msg[b], agg_sh.at[idx_v.at[b, 1]],
                                  sem_s[b]).wait()

        def compute(b):
            @plsc.parallel_loop(0, CHUNK // 2, unroll=4)
            def _(p):
                _relu_add_rows(msg[b], ebuf[b], p)

        start_loads(0, 0)
        start_loads(1, 1)
        wait_idx(0, 0)
        start_gather(0)

        def pair_body(kk, carry):
            g = 2 * kk
            wait_idx(g + 1, 1)

            @pl.when(kk > 0)
            def _():
                wait_scatter(1)

            start_gather(1)
            wait_e(0)
            wait_gather(0)
            compute(0)
            start_scatter(0)

            @pl.when(kk < (NFULL // 2) - 1)
            def _():
                start_loads(g + 2, 0)

            @pl.when(kk < (NFULL // 2) - 1)
            def _():
                wait_idx(g + 2, 0)
                wait_scatter(0)
                start_gather(0)

            wait_e(1)
            wait_gather(1)
            compute(1)
            start_scatter(1)

            @pl.when(kk < (NFULL // 2) - 1)
            def _():
                start_loads(g + 3, 1)

            return carry

        lax.fori_loop(0, NFULL // 2, pair_body, 0)
        wait_scatter(0)
        wait_scatter(1)

        if TAIL:
            base = base0 + NFULL * CHUNK
            pltpu.sync_copy(src_hbm.at[pl.ds(base, TAIL)], srct_v)
            pltpu.sync_copy(dst_hbm.at[pl.ds(base, TAIL)], dstt_v)
            pbase = wid * (EDGES_PER_W // 2) + NFULL * (CHUNK // 2)
            pltpu.sync_copy(e_hbm.at[pl.ds(pbase, TAIL // 2)],
                            e0.at[pl.ds(0, TAIL // 2)])
            pltpu.async_copy(h_hbm.at[srct_v], msg0.at[pl.ds(0, TAIL)],
                             sg0).wait()

            @plsc.parallel_loop(0, TAIL // 2, unroll=2)
            def _(p):
                _relu_add_rows(msg0, e0, p)
            pltpu.sync_copy(msg0.at[pl.ds(0, TAIL)], agg_sh.at[dstt_v],
                            add=True)

        plsc.subcore_barrier()
        pltpu.sync_copy(agg_sh.at[stripe], out_hbm.at[cid, stripe])

        @pl.when(sid == 0)
        def _():
            pltpu.sync_copy(agg_sh.at[tail_stripe],
                            out_hbm.at[cid, tail_stripe])

    kernel_body()


def _mlp_body(eps_ref, h_ref, agg_ref, w1_ref, b1_ref, w2_ref, b2_ref,
              batch_ref, hout_ref, pool_ref):
    eps = eps_ref[0, 0]
    z = h_ref[...] * (1.0 + eps) + agg_ref[0] + agg_ref[1]
    t = jnp.maximum(
        jax.lax.dot_general(z, w1_ref[...], (((1,), (0,)), ((), ())),
                            preferred_element_type=jnp.float32)
        + b1_ref[0][None, :], 0.0)
    o = jnp.maximum(
        jax.lax.dot_general(t, w2_ref[...], (((1,), (0,)), ((), ())),
                            preferred_element_type=jnp.float32)
        + b2_ref[0][None, :], 0.0)
    hout_ref[...] = o
    b = batch_ref[0, 0, :]
    oh = (b[:, None] == lax.broadcasted_iota(jnp.int32, (NODE_BLK, N_GRAPHS), 1)
          ).astype(jnp.float32)
    contrib = jax.lax.dot_general(oh, o, (((0,), (0,)), ((), ())),
                                  preferred_element_type=jnp.float32)

    @pl.when(pl.program_id(0) == 0)
    def _():
        pool_ref[...] = contrib

    @pl.when(pl.program_id(0) != 0)
    def _():
        pool_ref[...] = pool_ref[...] + contrib


def _mlp_layer(eps, h, agg2, w1, b1, w2, b2, batch3):
    return pl.pallas_call(
        _mlp_body,
        grid=(NODE_GRID,),
        in_specs=[
            pl.BlockSpec(memory_space=pltpu.SMEM),
            pl.BlockSpec((NODE_BLK, NHID), lambda i: (i, 0)),
            pl.BlockSpec((SC_CORES, NODE_BLK, NHID), lambda i: (0, i, 0)),
            pl.BlockSpec((NHID, 2 * NHID), lambda i: (0, 0)),
            pl.BlockSpec((1, 2 * NHID), lambda i: (0, 0)),
            pl.BlockSpec((2 * NHID, NHID), lambda i: (0, 0)),
            pl.BlockSpec((1, NHID), lambda i: (0, 0)),
            pl.BlockSpec((1, 1, NODE_BLK), lambda i: (i, 0, 0)),
        ],
        out_specs=[
            pl.BlockSpec((NODE_BLK, NHID), lambda i: (i, 0)),
            pl.BlockSpec((N_GRAPHS, NHID), lambda i: (0, 0)),
        ],
        out_shape=[
            jax.ShapeDtypeStruct((N_NODES, NHID), jnp.float32),
            jax.ShapeDtypeStruct((N_GRAPHS, NHID), jnp.float32),
        ],
    )(eps, h, agg2, w1, b1, w2, b2, batch3)


def kernel(x, edge_index, edge_attr, batch, atom_tables, bond_tables,
           mlp_params):
    atom_tab = jnp.zeros((ATOM_PAD, NHID), jnp.float32)
    atom_tab = atom_tab.at[:ATOM_TOT].set(jnp.concatenate(atom_tables, axis=0))
    bond_tab = jnp.zeros((BOND_PAD, NHID), jnp.float32)
    bond_tab = bond_tab.at[:BOND_TOT].set(jnp.concatenate(bond_tables, axis=0))

    h = _atom_encoder(x, atom_tab)
    e16 = _bond_encoder(edge_attr, bond_tab)

    src = edge_index[0]
    dst = edge_index[1]
    zeros = jnp.zeros((N_NODES, NHID), jnp.float32)
    batch3 = batch.reshape(NODE_GRID, 1, NODE_BLK)

    hs = []
    pools = []
    for (w1, b1, w2, b2, eps) in mlp_params:
        agg2 = _edge_agg(h, e16, src, dst, zeros)
        h, pool = _mlp_layer(
            jnp.reshape(eps, (1, 1)), h, agg2, w1,
            jnp.reshape(b1, (1, 2 * NHID)), w2, jnp.reshape(b2, (1, NHID)),
            batch3)
        hs.append(h)
        pools.append(pool)

    node_embs = jnp.concatenate(hs, axis=-1)
    graph_embs = jnp.concatenate(pools, axis=-1)
    return (graph_embs, node_embs)


# speedup vs baseline: 1.7793x; 1.1081x over previous
import functools

import jax
import jax.numpy as jnp
from jax import lax
from jax.experimental import pallas as pl
from jax.experimental.pallas import tpu as pltpu
from jax.experimental.pallas import tpu_sc as plsc

NHID = 128
NLAYER = 5
N_NODES = 10000
N_EDGES = 320000
N_GRAPHS = 128

ATOM_DIMS = [119, 4, 12, 12, 10, 6, 6, 2, 2, 60]
BOND_DIMS = [5, 6, 2, 22]
ATOM_ROWS = [d + 2 for d in ATOM_DIMS]
BOND_ROWS = [d + 1 for d in BOND_DIMS]
ATOM_OFF = [sum(ATOM_ROWS[:i]) for i in range(len(ATOM_ROWS))]
BOND_OFF = [sum(BOND_ROWS[:i]) for i in range(len(BOND_ROWS))]
ATOM_TOT = sum(ATOM_ROWS)
BOND_TOT = sum(BOND_ROWS)
ATOM_PAD = 256
BOND_PAD = 40

SC_CORES = 2
SC_SUBCORES = 16
NW = SC_CORES * SC_SUBCORES
EDGES_PER_W = N_EDGES // NW
CHUNK = 128
NFULL = EDGES_PER_W // CHUNK
TAIL = EDGES_PER_W - NFULL * CHUNK
ROWS_PER_SUB = (N_NODES // SC_SUBCORES) // 8 * 8
ROWS_TAIL = N_NODES - ROWS_PER_SUB * SC_SUBCORES

NODE_BLK = 2000
NODE_GRID = N_NODES // NODE_BLK
EDGE_BLK = 4000
EDGE_GRID = N_EDGES // EDGE_BLK


def _atom_encoder_body(x_ref, tab_ref, out_ref):
    xb = x_ref[...]
    col = lax.broadcasted_iota(jnp.int32, (NODE_BLK, ATOM_PAD), 1)
    oh = jnp.zeros((NODE_BLK, ATOM_PAD), jnp.float32)
    for i, off in enumerate(ATOM_OFF):
        oh = oh + (col == xb[:, i][:, None] + off).astype(jnp.float32)
    h = jax.lax.dot_general(
        oh, tab_ref[...], (((1,), (0,)), ((), ())),
        preferred_element_type=jnp.float32)
    out_ref[...] = h


def _bond_encoder_body(a_ref, tab_ref, outpk_ref):
    ab = a_ref[...]
    col = lax.broadcasted_iota(jnp.int32, (EDGE_BLK, BOND_PAD), 1)
    oh = jnp.zeros((EDGE_BLK, BOND_PAD), jnp.float32)
    for i, off in enumerate(BOND_OFF):
        oh = oh + (col == ab[:, i][:, None] + off).astype(jnp.float32)
    e = jax.lax.dot_general(
        oh, tab_ref[...], (((1,), (0,)), ((), ())),
        preferred_element_type=jnp.float32)
    er = jnp.reshape(e.astype(jnp.bfloat16).astype(jnp.float32),
                     (EDGE_BLK // 2, 2, NHID))
    b_even = lax.bitcast_convert_type(er[:, 0, :], jnp.int32)
    b_odd = lax.bitcast_convert_type(er[:, 1, :], jnp.int32)
    outpk_ref[...] = jnp.bitwise_or(
        lax.shift_right_logical(b_even, 16), b_odd)


def _atom_encoder(x, atom_tab):
    return pl.pallas_call(
        _atom_encoder_body,
        grid=(NODE_GRID,),
        in_specs=[
            pl.BlockSpec((NODE_BLK, len(ATOM_DIMS)), lambda i: (i, 0)),
            pl.BlockSpec((ATOM_PAD, NHID), lambda i: (0, 0)),
        ],
        out_specs=pl.BlockSpec((NODE_BLK, NHID), lambda i: (i, 0)),
        out_shape=jax.ShapeDtypeStruct((N_NODES, NHID), jnp.float32),
    )(x, atom_tab)


def _bond_encoder(edge_attr, bond_tab):
    return pl.pallas_call(
        _bond_encoder_body,
        grid=(EDGE_GRID,),
        in_specs=[
            pl.BlockSpec((EDGE_BLK, len(BOND_DIMS)), lambda i: (i, 0)),
            pl.BlockSpec((BOND_PAD, NHID), lambda i: (0, 0)),
        ],
        out_specs=pl.BlockSpec((EDGE_BLK // 2, NHID), lambda i: (i, 0)),
        out_shape=jax.ShapeDtypeStruct((N_EDGES // 2, NHID), jnp.int32),
    )(edge_attr, bond_tab)


def _decode_pair(w):
    lo = lax.bitcast_convert_type(lax.shift_left(w, 16), jnp.float32)
    hi = lax.bitcast_convert_type(jnp.bitwise_and(w, -65536), jnp.float32)
    return lo, hi


def _relu_add_rows(msg_v, eview, p):
    i0 = 2 * p
    for j in range(NHID // 16):
        sl = pl.ds(j * 16, 16)
        ew = eview[p, sl]
        e_lo, e_hi = _decode_pair(ew)
        msg_v[i0, sl] = jnp.maximum(msg_v[i0, sl] + e_lo, 0.0)
        msg_v[i0 + 1, sl] = jnp.maximum(msg_v[i0 + 1, sl] + e_hi, 0.0)


@functools.partial(
    pl.kernel,
    out_type=jax.ShapeDtypeStruct((SC_CORES, N_NODES, NHID), jnp.float32),
    mesh=plsc.VectorSubcoreMesh(core_axis_name="c", subcore_axis_name="s"),
    scratch_types=[
        pltpu.VMEM((2, 2, CHUNK), jnp.int32),
        pltpu.VMEM((TAIL,), jnp.int32),
        pltpu.VMEM((TAIL,), jnp.int32),
        pltpu.VMEM((CHUNK, NHID), jnp.float32),
        pltpu.VMEM((CHUNK, NHID), jnp.float32),
        pltpu.VMEM((CHUNK // 2, NHID), jnp.int32),
        pltpu.VMEM((CHUNK // 2, NHID), jnp.int32),
        pltpu.VMEM_SHARED((N_NODES, NHID), jnp.float32),
        pltpu.SemaphoreType.DMA,
        pltpu.SemaphoreType.DMA,
        pltpu.SemaphoreType.DMA,
        pltpu.SemaphoreType.DMA,
        pltpu.SemaphoreType.DMA,
        pltpu.SemaphoreType.DMA,
        pltpu.SemaphoreType.DMA,
        pltpu.SemaphoreType.DMA,
    ],
)
def _edge_agg(h_hbm, e_hbm, src_hbm, dst_hbm, zeros_hbm, out_hbm,
              idx_v, srct_v, dstt_v, msg0, msg1, e0, e1, agg_sh,
              si0, si1, se0, se1, sg0, sg1, ss0, ss1):
    cid = lax.axis_index("c")
    sid = lax.axis_index("s")
    wid = sid * SC_CORES + cid
    stripe = pl.ds(sid * ROWS_PER_SUB, ROWS_PER_SUB)
    tail_stripe = pl.ds(SC_SUBCORES * ROWS_PER_SUB, ROWS_TAIL)

    msg = (msg0, msg1)
    ebuf = (e0, e1)
    sem_i = (si0, si1)
    sem_e = (se0, se1)
    sem_g = (sg0, sg1)
    sem_s = (ss0, ss1)

    def kernel_body():
        pltpu.sync_copy(zeros_hbm.at[stripe], agg_sh.at[stripe])

        @pl.when(sid == 0)
        def _():
            pltpu.sync_copy(zeros_hbm.at[tail_stripe], agg_sh.at[tail_stripe])

        plsc.subcore_barrier()

        base0 = wid * EDGES_PER_W

        def start_loads(g, b):
            base = base0 + g * CHUNK
            pbase = wid * (EDGES_PER_W // 2) + g * (CHUNK // 2)
            pltpu.async_copy(src_hbm.at[pl.ds(base, CHUNK)],
                             idx_v.at[b, 0], sem_i[b])
            pltpu.async_copy(dst_hbm.at[pl.ds(base, CHUNK)],
                             idx_v.at[b, 1], sem_i[b])
            pltpu.async_copy(e_hbm.at[pl.ds(pbase, CHUNK // 2)], ebuf[b],
                             sem_e[b])

        def wait_idx(g, b):
            base = base0 + g * CHUNK
            pltpu.make_async_copy(src_hbm.at[pl.ds(base, CHUNK)],
                                  idx_v.at[b, 0], sem_i[b]).wait()
            pltpu.make_async_copy(dst_hbm.at[pl.ds(base, CHUNK)],
                                  idx_v.at[b, 1], sem_i[b]).wait()

        def wait_e(b):
            pltpu.make_async_copy(e_hbm.at[pl.ds(0, CHUNK // 2)],
                                  ebuf[b], sem_e[b]).wait()

        def start_gather(b):
            pltpu.async_copy(h_hbm.at[idx_v.at[b, 0]], msg[b], sem_g[b])

        def wait_gather(b):
            pltpu.make_async_copy(h_hbm.at[idx_v.at[b, 0]],
                                  msg[b], sem_g[b]).wait()

        def start_scatter(b):
            pltpu.async_copy(msg[b], agg_sh.at[idx_v.at[b, 1]], sem_s[b],
                             add=True)

        def wait_scatter(b):
            pltpu.make_async_copy(msg[b], agg_sh.at[idx_v.at[b, 1]],
                                  sem_s[b]).wait()

        def compute(b):
            @plsc.parallel_loop(0, CHUNK // 2, unroll=4)
            def _(p):
                _relu_add_rows(msg[b], ebuf[b], p)

        start_loads(0, 0)
        start_loads(1, 1)
        wait_idx(0, 0)
        start_gather(0)

        def pair_body(kk, carry):
            g = 2 * kk
            wait_idx(g + 1, 1)

            @pl.when(kk > 0)
            def _():
                wait_scatter(1)

            start_gather(1)
            wait_e(0)
            wait_gather(0)
            compute(0)
            start_scatter(0)

            @pl.when(kk < (NFULL // 2) - 1)
            def _():
                start_loads(g + 2, 0)

            @pl.when(kk < (NFULL // 2) - 1)
            def _():
                wait_idx(g + 2, 0)
                wait_scatter(0)
                start_gather(0)

            wait_e(1)
            wait_gather(1)
            compute(1)
            start_scatter(1)

            @pl.when(kk < (NFULL // 2) - 1)
            def _():
                start_loads(g + 3, 1)

            return carry

        lax.fori_loop(0, NFULL // 2, pair_body, 0)
        wait_scatter(0)
        wait_scatter(1)

        if TAIL:
            base = base0 + NFULL * CHUNK
            pltpu.sync_copy(src_hbm.at[pl.ds(base, TAIL)], srct_v)
            pltpu.sync_copy(dst_hbm.at[pl.ds(base, TAIL)], dstt_v)
            pbase = wid * (EDGES_PER_W // 2) + NFULL * (CHUNK // 2)
            pltpu.sync_copy(e_hbm.at[pl.ds(pbase, TAIL // 2)],
                            e0.at[pl.ds(0, TAIL // 2)])
            pltpu.async_copy(h_hbm.at[srct_v], msg0.at[pl.ds(0, TAIL)],
                             sg0).wait()

            @plsc.parallel_loop(0, TAIL // 2, unroll=2)
            def _(p):
                _relu_add_rows(msg0, e0, p)
            pltpu.sync_copy(msg0.at[pl.ds(0, TAIL)], agg_sh.at[dstt_v],
                            add=True)

        plsc.subcore_barrier()
        pltpu.sync_copy(agg_sh.at[stripe], out_hbm.at[cid, stripe])

        @pl.when(sid == 0)
        def _():
            pltpu.sync_copy(agg_sh.at[tail_stripe],
                            out_hbm.at[cid, tail_stripe])

    kernel_body()


def _mlp_body(eps_ref, h_ref, agg_ref, w1_ref, b1_ref, w2_ref, b2_ref,
              batch_ref, hout_ref, pool_ref):
    eps = eps_ref[0, 0]
    z = h_ref[...] * (1.0 + eps) + agg_ref[0] + agg_ref[1]
    t = jnp.maximum(
        jax.lax.dot_general(z, w1_ref[...], (((1,), (0,)), ((), ())),
                            preferred_element_type=jnp.float32)
        + b1_ref[0][None, :], 0.0)
    o = jnp.maximum(
        jax.lax.dot_general(t, w2_ref[...], (((1,), (0,)), ((), ())),
                            preferred_element_type=jnp.float32)
        + b2_ref[0][None, :], 0.0)
    hout_ref[...] = o
    b = batch_ref[0, 0, :]
    oh = (b[:, None] == lax.broadcasted_iota(jnp.int32, (NODE_BLK, N_GRAPHS), 1)
          ).astype(jnp.float32)
    contrib = jax.lax.dot_general(oh, o, (((0,), (0,)), ((), ())),
                                  preferred_element_type=jnp.float32)

    @pl.when(pl.program_id(0) == 0)
    def _():
        pool_ref[...] = contrib

    @pl.when(pl.program_id(0) != 0)
    def _():
        pool_ref[...] = pool_ref[...] + contrib


def _mlp_layer(eps, h, agg2, w1, b1, w2, b2, batch3):
    return pl.pallas_call(
        _mlp_body,
        grid=(NODE_GRID,),
        in_specs=[
            pl.BlockSpec(memory_space=pltpu.SMEM),
            pl.BlockSpec((NODE_BLK, NHID), lambda i: (i, 0)),
            pl.BlockSpec((SC_CORES, NODE_BLK, NHID), lambda i: (0, i, 0)),
            pl.BlockSpec((NHID, 2 * NHID), lambda i: (0, 0)),
            pl.BlockSpec((1, 2 * NHID), lambda i: (0, 0)),
            pl.BlockSpec((2 * NHID, NHID), lambda i: (0, 0)),
            pl.BlockSpec((1, NHID), lambda i: (0, 0)),
            pl.BlockSpec((1, 1, NODE_BLK), lambda i: (i, 0, 0)),
        ],
        out_specs=[
            pl.BlockSpec((NODE_BLK, NHID), lambda i: (i, 0)),
            pl.BlockSpec((N_GRAPHS, NHID), lambda i: (0, 0)),
        ],
        out_shape=[
            jax.ShapeDtypeStruct((N_NODES, NHID), jnp.float32),
            jax.ShapeDtypeStruct((N_GRAPHS, NHID), jnp.float32),
        ],
    )(eps, h, agg2, w1, b1, w2, b2, batch3)


def kernel(x, edge_index, edge_attr, batch, atom_tables, bond_tables,
           mlp_params):
    atom_tab = jnp.zeros((ATOM_PAD, NHID), jnp.float32)
    atom_tab = atom_tab.at[:ATOM_TOT].set(jnp.concatenate(atom_tables, axis=0))
    bond_tab = jnp.zeros((BOND_PAD, NHID), jnp.float32)
    bond_tab = bond_tab.at[:BOND_TOT].set(jnp.concatenate(bond_tables, axis=0))

    h = _atom_encoder(x, atom_tab)
    e16 = _bond_encoder(edge_attr, bond_tab)

    src = edge_index[0]
    dst = edge_index[1]
    zeros = jnp.zeros((N_NODES, NHID), jnp.float32)
    batch3 = batch.reshape(NODE_GRID, 1, NODE_BLK)

    hs = []
    pools = []
    for (w1, b1, w2, b2, eps) in mlp_params:
        agg2 = _edge_agg(h, e16, src, dst, zeros)
        h, pool = _mlp_layer(
            jnp.reshape(eps, (1, 1)), h, agg2, w1,
            jnp.reshape(b1, (1, 2 * NHID)), w2, jnp.reshape(b2, (1, NHID)),
            batch3)
        hs.append(h)
        pools.append(pool)

    node_embs = jnp.concatenate(hs, axis=-1)
    graph_embs = jnp.concatenate(pools, axis=-1)
    return (graph_embs, node_embs)


# speedup vs baseline: 1.9426x; 1.0918x over previous
import functools

import jax
import jax.numpy as jnp
from jax import lax
from jax.experimental import pallas as pl
from jax.experimental.pallas import tpu as pltpu
from jax.experimental.pallas import tpu_sc as plsc

NHID = 128
NLAYER = 5
N_NODES = 10000
N_EDGES = 320000
N_GRAPHS = 128

ATOM_DIMS = [119, 4, 12, 12, 10, 6, 6, 2, 2, 60]
BOND_DIMS = [5, 6, 2, 22]
ATOM_ROWS = [d + 2 for d in ATOM_DIMS]
BOND_ROWS = [d + 1 for d in BOND_DIMS]
ATOM_OFF = [sum(ATOM_ROWS[:i]) for i in range(len(ATOM_ROWS))]
BOND_OFF = [sum(BOND_ROWS[:i]) for i in range(len(BOND_ROWS))]
ATOM_TOT = sum(ATOM_ROWS)
BOND_TOT = sum(BOND_ROWS)
ATOM_PAD = 256
BOND_PAD = 40

SC_CORES = 2
SC_SUBCORES = 16
NW = SC_CORES * SC_SUBCORES
CHUNK = 128
HALF = CHUNK // 2
NCHUNKS = N_EDGES // CHUNK
CH_PER_W = NCHUNKS // NW
LEFTOVER = NCHUNKS - CH_PER_W * NW
ROWS_PER_SUB = (N_NODES // SC_SUBCORES) // 8 * 8
ROWS_TAIL = N_NODES - ROWS_PER_SUB * SC_SUBCORES

NODE_BLK = 2000
NODE_GRID = N_NODES // NODE_BLK
EDGE_BLK = 16000
EDGE_GRID = N_EDGES // EDGE_BLK


def _atom_encoder_body(x_ref, tab_ref, out_ref):
    xb = x_ref[...]
    col = lax.broadcasted_iota(jnp.int32, (N_NODES, ATOM_PAD), 1)
    oh = jnp.zeros((N_NODES, ATOM_PAD), jnp.float32)
    for i, off in enumerate(ATOM_OFF):
        oh = oh + (col == xb[:, i][:, None] + off).astype(jnp.float32)
    h = jax.lax.dot_general(
        oh, tab_ref[...], (((1,), (0,)), ((), ())),
        preferred_element_type=jnp.float32)
    out_ref[...] = h


def _bond_encoder_body(a_ref, tab_ref, outpk_ref):
    ab = a_ref[...]
    col = lax.broadcasted_iota(jnp.int32, (EDGE_BLK, BOND_PAD), 1)
    oh = jnp.zeros((EDGE_BLK, BOND_PAD), jnp.float32)
    for i, off in enumerate(BOND_OFF):
        oh = oh + (col == ab[:, i][:, None] + off).astype(jnp.float32)
    e = jax.lax.dot_general(
        oh, tab_ref[...], (((1,), (0,)), ((), ())),
        preferred_element_type=jnp.float32)
    er = jnp.reshape(e.astype(jnp.bfloat16).astype(jnp.float32),
                     (EDGE_BLK // CHUNK, CHUNK, NHID))
    b_lo = lax.bitcast_convert_type(er[:, :HALF, :], jnp.int32)
    b_hi = lax.bitcast_convert_type(er[:, HALF:, :], jnp.int32)
    w = jnp.bitwise_or(lax.shift_right_logical(b_lo, 16), b_hi)
    outpk_ref[...] = jnp.reshape(w, (EDGE_BLK // 2, NHID))


def _atom_encoder(x, atom_tab):
    return pl.pallas_call(
        _atom_encoder_body,
        grid=(1,),
        in_specs=[
            pl.BlockSpec((N_NODES, len(ATOM_DIMS)), lambda i: (0, 0)),
            pl.BlockSpec((ATOM_PAD, NHID), lambda i: (0, 0)),
        ],
        out_specs=pl.BlockSpec((N_NODES, NHID), lambda i: (0, 0)),
        out_shape=jax.ShapeDtypeStruct((N_NODES, NHID), jnp.float32),
    )(x, atom_tab)


def _bond_encoder(edge_attr, bond_tab):
    return pl.pallas_call(
        _bond_encoder_body,
        grid=(EDGE_GRID,),
        in_specs=[
            pl.BlockSpec((EDGE_BLK, len(BOND_DIMS)), lambda i: (i, 0)),
            pl.BlockSpec((BOND_PAD, NHID), lambda i: (0, 0)),
        ],
        out_specs=pl.BlockSpec((EDGE_BLK // 2, NHID), lambda i: (i, 0)),
        out_shape=jax.ShapeDtypeStruct((N_EDGES // 2, NHID), jnp.int32),
    )(edge_attr, bond_tab)


def _decode_pair(w):
    lo = lax.bitcast_convert_type(lax.shift_left(w, 16), jnp.float32)
    hi = lax.bitcast_convert_type(jnp.bitwise_and(w, -65536), jnp.float32)
    return lo, hi


def _relu_add_rows(msg_v, eview, p):
    for j in range(NHID // 16):
        sl = pl.ds(j * 16, 16)
        ew = eview[p, sl]
        e_lo, e_hi = _decode_pair(ew)
        msg_v[p, sl] = jnp.maximum(msg_v[p, sl] + e_lo, 0.0)
        msg_v[p + HALF, sl] = jnp.maximum(msg_v[p + HALF, sl] + e_hi, 0.0)


@functools.partial(
    pl.kernel,
    out_type=jax.ShapeDtypeStruct((SC_CORES, N_NODES, NHID), jnp.float32),
    mesh=plsc.VectorSubcoreMesh(core_axis_name="c", subcore_axis_name="s"),
    scratch_types=[
        pltpu.VMEM((2, 2, CHUNK), jnp.int32),
        pltpu.VMEM((CHUNK, NHID), jnp.float32),
        pltpu.VMEM((CHUNK, NHID), jnp.float32),
        pltpu.VMEM((HALF, NHID), jnp.int32),
        pltpu.VMEM((HALF, NHID), jnp.int32),
        pltpu.VMEM_SHARED((N_NODES, NHID), jnp.float32),
        pltpu.SemaphoreType.DMA,
        pltpu.SemaphoreType.DMA,
        pltpu.SemaphoreType.DMA,
        pltpu.SemaphoreType.DMA,
        pltpu.SemaphoreType.DMA,
        pltpu.SemaphoreType.DMA,
        pltpu.SemaphoreType.DMA,
        pltpu.SemaphoreType.DMA,
    ],
)
def _edge_agg(h_hbm, e_hbm, src_hbm, dst_hbm, zeros_hbm, out_hbm,
              idx_v, msg0, msg1, e0, e1, agg_sh,
              si0, si1, se0, se1, sg0, sg1, ss0, ss1):
    cid = lax.axis_index("c")
    sid = lax.axis_index("s")
    wid = sid * SC_CORES + cid
    stripe = pl.ds(sid * ROWS_PER_SUB, ROWS_PER_SUB)
    tail_stripe = pl.ds(SC_SUBCORES * ROWS_PER_SUB, ROWS_TAIL)

    msg = (msg0, msg1)
    ebuf = (e0, e1)
    sem_i = (si0, si1)
    sem_e = (se0, se1)
    sem_g = (sg0, sg1)
    sem_s = (ss0, ss1)

    def kernel_body():
        pltpu.sync_copy(zeros_hbm.at[stripe], agg_sh.at[stripe])

        @pl.when(sid == 0)
        def _():
            pltpu.sync_copy(zeros_hbm.at[tail_stripe], agg_sh.at[tail_stripe])

        plsc.subcore_barrier()

        def start_loads(g, b):
            c = wid + NW * g
            base = c * CHUNK
            pltpu.async_copy(src_hbm.at[pl.ds(base, CHUNK)],
                             idx_v.at[b, 0], sem_i[b])
            pltpu.async_copy(dst_hbm.at[pl.ds(base, CHUNK)],
                             idx_v.at[b, 1], sem_i[b])
            pltpu.async_copy(e_hbm.at[pl.ds(c * HALF, HALF)], ebuf[b],
                             sem_e[b])

        def wait_idx(g, b):
            base = (wid + NW * g) * CHUNK
            pltpu.make_async_copy(src_hbm.at[pl.ds(base, CHUNK)],
                                  idx_v.at[b, 0], sem_i[b]).wait()
            pltpu.make_async_copy(dst_hbm.at[pl.ds(base, CHUNK)],
                                  idx_v.at[b, 1], sem_i[b]).wait()

        def wait_e(b):
            pltpu.make_async_copy(e_hbm.at[pl.ds(0, HALF)],
                                  ebuf[b], sem_e[b]).wait()

        def start_gather(b):
            pltpu.async_copy(h_hbm.at[idx_v.at[b, 0]], msg[b], sem_g[b])

        def wait_gather(b):
            pltpu.make_async_copy(h_hbm.at[idx_v.at[b, 0]],
                                  msg[b], sem_g[b]).wait()

        def start_scatter(b):
            pltpu.async_copy(msg[b], agg_sh.at[idx_v.at[b, 1]], sem_s[b],
                             add=True)

        def wait_scatter(b):
            pltpu.make_async_copy(msg[b], agg_sh.at[idx_v.at[b, 1]],
                                  sem_s[b]).wait()

        def compute(b):
            @plsc.parallel_loop(0, CHUNK // 2, unroll=4)
            def _(p):
                _relu_add_rows(msg[b], ebuf[b], p)

        start_loads(0, 0)
        start_loads(1, 1)
        wait_idx(0, 0)
        start_gather(0)

        def pair_body(kk, carry):
            g = 2 * kk
            wait_idx(g + 1, 1)

            @pl.when(kk > 0)
            def _():
                wait_scatter(1)

            start_gather(1)
            wait_e(0)
            wait_gather(0)
            compute(0)
            start_scatter(0)

            @pl.when(kk < (CH_PER_W // 2) - 1)
            def _():
                start_loads(g + 2, 0)

            @pl.when(kk < (CH_PER_W // 2) - 1)
            def _():
                wait_idx(g + 2, 0)
                wait_scatter(0)
                start_gather(0)

            wait_e(1)
            wait_gather(1)
            compute(1)
            start_scatter(1)

            @pl.when(kk < (CH_PER_W // 2) - 1)
            def _():
                start_loads(g + 3, 1)

            return carry

        lax.fori_loop(0, CH_PER_W // 2, pair_body, 0)
        wait_scatter(0)
        wait_scatter(1)

        @pl.when(wid < LEFTOVER)
        def _():
            c = NW * CH_PER_W + wid
            base = c * CHUNK
            pltpu.sync_copy(src_hbm.at[pl.ds(base, CHUNK)], idx_v.at[0, 0])
            pltpu.sync_copy(dst_hbm.at[pl.ds(base, CHUNK)], idx_v.at[0, 1])
            pltpu.sync_copy(e_hbm.at[pl.ds(c * HALF, HALF)], e0)
            pltpu.async_copy(h_hbm.at[idx_v.at[0, 0]], msg0, sg0).wait()
            compute(0)
            pltpu.sync_copy(msg0, agg_sh.at[idx_v.at[0, 1]], add=True)

        plsc.subcore_barrier()
        pltpu.sync_copy(agg_sh.at[stripe], out_hbm.at[cid, stripe])

        @pl.when(sid == 0)
        def _():
            pltpu.sync_copy(agg_sh.at[tail_stripe],
                            out_hbm.at[cid, tail_stripe])

    kernel_body()


def _mlp_body(eps_ref, h_ref, agg_ref, w1_ref, b1_ref, w2_ref, b2_ref,
              batch_ref, hout_ref, pool_ref):
    eps = eps_ref[0, 0]
    z = h_ref[...] * (1.0 + eps) + agg_ref[0] + agg_ref[1]
    t = jnp.maximum(
        jax.lax.dot_general(z, w1_ref[...], (((1,), (0,)), ((), ())),
                            preferred_element_type=jnp.float32)
        + b1_ref[0][None, :], 0.0)
    o = jnp.maximum(
        jax.lax.dot_general(t, w2_ref[...], (((1,), (0,)), ((), ())),
                            preferred_element_type=jnp.float32)
        + b2_ref[0][None, :], 0.0)
    hout_ref[...] = o
    b = batch_ref[0, 0, :]
    oh = (b[:, None] == lax.broadcasted_iota(jnp.int32, (NODE_BLK, N_GRAPHS), 1)
          ).astype(jnp.float32)
    contrib = jax.lax.dot_general(oh, o, (((0,), (0,)), ((), ())),
                                  preferred_element_type=jnp.float32)

    @pl.when(pl.program_id(0) == 0)
    def _():
        pool_ref[...] = contrib

    @pl.when(pl.program_id(0) != 0)
    def _():
        pool_ref[...] = pool_ref[...] + contrib


def _mlp_layer(eps, h, agg2, w1, b1, w2, b2, batch3):
    return pl.pallas_call(
        _mlp_body,
        grid=(NODE_GRID,),
        in_specs=[
            pl.BlockSpec(memory_space=pltpu.SMEM),
            pl.BlockSpec((NODE_BLK, NHID), lambda i: (i, 0)),
            pl.BlockSpec((SC_CORES, NODE_BLK, NHID), lambda i: (0, i, 0)),
            pl.BlockSpec((NHID, 2 * NHID), lambda i: (0, 0)),
            pl.BlockSpec((1, 2 * NHID), lambda i: (0, 0)),
            pl.BlockSpec((2 * NHID, NHID), lambda i: (0, 0)),
            pl.BlockSpec((1, NHID), lambda i: (0, 0)),
            pl.BlockSpec((1, 1, NODE_BLK), lambda i: (i, 0, 0)),
        ],
        out_specs=[
            pl.BlockSpec((NODE_BLK, NHID), lambda i: (i, 0)),
            pl.BlockSpec((N_GRAPHS, NHID), lambda i: (0, 0)),
        ],
        out_shape=[
            jax.ShapeDtypeStruct((N_NODES, NHID), jnp.float32),
            jax.ShapeDtypeStruct((N_GRAPHS, NHID), jnp.float32),
        ],
    )(eps, h, agg2, w1, b1, w2, b2, batch3)


def kernel(x, edge_index, edge_attr, batch, atom_tables, bond_tables,
           mlp_params):
    atom_tab = jnp.zeros((ATOM_PAD, NHID), jnp.float32)
    atom_tab = atom_tab.at[:ATOM_TOT].set(jnp.concatenate(atom_tables, axis=0))
    bond_tab = jnp.zeros((BOND_PAD, NHID), jnp.float32)
    bond_tab = bond_tab.at[:BOND_TOT].set(jnp.concatenate(bond_tables, axis=0))

    h = _atom_encoder(x, atom_tab)
    e16 = _bond_encoder(edge_attr, bond_tab)

    src = edge_index[0]
    dst = edge_index[1]
    zeros = jnp.zeros((N_NODES, NHID), jnp.float32)
    batch3 = batch.reshape(NODE_GRID, 1, NODE_BLK)

    hs = []
    pools = []
    for (w1, b1, w2, b2, eps) in mlp_params:
        agg2 = _edge_agg(h, e16, src, dst, zeros)
        h, pool = _mlp_layer(
            jnp.reshape(eps, (1, 1)), h, agg2, w1,
            jnp.reshape(b1, (1, 2 * NHID)), w2, jnp.reshape(b2, (1, NHID)),
            batch3)
        hs.append(h)
        pools.append(pool)

    node_embs = jnp.concatenate(hs, axis=-1)
    graph_embs = jnp.concatenate(pools, axis=-1)
    return (graph_embs, node_embs)


# speedup vs baseline: 2.0786x; 1.0700x over previous
import functools

import jax
import jax.numpy as jnp
from jax import lax
from jax.experimental import pallas as pl
from jax.experimental.pallas import tpu as pltpu
from jax.experimental.pallas import tpu_sc as plsc

NHID = 128
NLAYER = 5
N_NODES = 10000
N_EDGES = 320000
N_GRAPHS = 128

ATOM_DIMS = [119, 4, 12, 12, 10, 6, 6, 2, 2, 60]
BOND_DIMS = [5, 6, 2, 22]
ATOM_ROWS = [d + 2 for d in ATOM_DIMS]
BOND_ROWS = [d + 1 for d in BOND_DIMS]
ATOM_OFF = [sum(ATOM_ROWS[:i]) for i in range(len(ATOM_ROWS))]
BOND_OFF = [sum(BOND_ROWS[:i]) for i in range(len(BOND_ROWS))]
ATOM_TOT = sum(ATOM_ROWS)
BOND_TOT = sum(BOND_ROWS)
ATOM_PAD = 256
BOND_PAD = 40

SC_CORES = 2
SC_SUBCORES = 16
NW = SC_CORES * SC_SUBCORES
CHUNK = 128
HALF = CHUNK // 2
NCHUNKS = N_EDGES // CHUNK
CH_PER_W = NCHUNKS // NW
LEFTOVER = NCHUNKS - CH_PER_W * NW
ROWS_PER_SUB = (N_NODES // SC_SUBCORES) // 8 * 8
ROWS_TAIL = N_NODES - ROWS_PER_SUB * SC_SUBCORES

NODE_BLK = 2000
NODE_GRID = N_NODES // NODE_BLK
EDGE_BLK = 32000
EDGE_GRID = N_EDGES // EDGE_BLK


def _atom_encoder_body(x_ref, tab_ref, out_ref):
    xb = x_ref[...]
    col = lax.broadcasted_iota(jnp.int32, (N_NODES, ATOM_PAD), 1)
    oh = jnp.zeros((N_NODES, ATOM_PAD), jnp.float32)
    for i, off in enumerate(ATOM_OFF):
        oh = oh + (col == xb[:, i][:, None] + off).astype(jnp.float32)
    h = jax.lax.dot_general(
        oh, tab_ref[...], (((1,), (0,)), ((), ())),
        preferred_element_type=jnp.float32)
    out_ref[...] = h


def _bond_encoder_body(a_ref, tab_ref, outpk_ref):
    ab = a_ref[...]
    col = lax.broadcasted_iota(jnp.int32, (EDGE_BLK, BOND_PAD), 1)
    oh = jnp.zeros((EDGE_BLK, BOND_PAD), jnp.float32)
    for i, off in enumerate(BOND_OFF):
        oh = oh + (col == ab[i, :][:, None] + off).astype(jnp.float32)
    e = jax.lax.dot_general(
        oh, tab_ref[...], (((1,), (0,)), ((), ())),
        preferred_element_type=jnp.float32)
    er = jnp.reshape(e.astype(jnp.bfloat16).astype(jnp.float32),
                     (EDGE_BLK // CHUNK, CHUNK, NHID))
    b_lo = lax.bitcast_convert_type(er[:, :HALF, :], jnp.int32)
    b_hi = lax.bitcast_convert_type(er[:, HALF:, :], jnp.int32)
    w = jnp.bitwise_or(lax.shift_right_logical(b_lo, 16), b_hi)
    outpk_ref[...] = jnp.reshape(w, (EDGE_BLK // 2, NHID))


def _atom_encoder(x, atom_tab):
    return pl.pallas_call(
        _atom_encoder_body,
        grid=(1,),
        in_specs=[
            pl.BlockSpec((N_NODES, len(ATOM_DIMS)), lambda i: (0, 0)),
            pl.BlockSpec((ATOM_PAD, NHID), lambda i: (0, 0)),
        ],
        out_specs=pl.BlockSpec((N_NODES, NHID), lambda i: (0, 0)),
        out_shape=jax.ShapeDtypeStruct((N_NODES, NHID), jnp.float32),
    )(x, atom_tab)


def _bond_encoder(edge_attr, bond_tab):
    return pl.pallas_call(
        _bond_encoder_body,
        grid=(EDGE_GRID,),
        in_specs=[
            pl.BlockSpec((len(BOND_DIMS), EDGE_BLK), lambda i: (0, i)),
            pl.BlockSpec((BOND_PAD, NHID), lambda i: (0, 0)),
        ],
        out_specs=pl.BlockSpec((EDGE_BLK // 2, NHID), lambda i: (i, 0)),
        out_shape=jax.ShapeDtypeStruct((N_EDGES // 2, NHID), jnp.int32),
    )(edge_attr, bond_tab)


def _decode_pair(w):
    lo = lax.bitcast_convert_type(lax.shift_left(w, 16), jnp.float32)
    hi = lax.bitcast_convert_type(jnp.bitwise_and(w, -65536), jnp.float32)
    return lo, hi


def _relu_add_rows(msg_v, eview, p):
    for j in range(NHID // 16):
        sl = pl.ds(j * 16, 16)
        ew = eview[p, sl]
        e_lo, e_hi = _decode_pair(ew)
        msg_v[p, sl] = jnp.maximum(msg_v[p, sl] + e_lo, 0.0)
        msg_v[p + HALF, sl] = jnp.maximum(msg_v[p + HALF, sl] + e_hi, 0.0)


@functools.partial(
    pl.kernel,
    out_type=jax.ShapeDtypeStruct((SC_CORES, N_NODES, NHID), jnp.float32),
    mesh=plsc.VectorSubcoreMesh(core_axis_name="c", subcore_axis_name="s"),
    scratch_types=[
        pltpu.VMEM((2, 2, CHUNK), jnp.int32),
        pltpu.VMEM((CHUNK, NHID), jnp.float32),
        pltpu.VMEM((CHUNK, NHID), jnp.float32),
        pltpu.VMEM((HALF, NHID), jnp.int32),
        pltpu.VMEM((HALF, NHID), jnp.int32),
        pltpu.VMEM_SHARED((N_NODES, NHID), jnp.float32),
        pltpu.SemaphoreType.DMA,
        pltpu.SemaphoreType.DMA,
        pltpu.SemaphoreType.DMA,
        pltpu.SemaphoreType.DMA,
        pltpu.SemaphoreType.DMA,
        pltpu.SemaphoreType.DMA,
        pltpu.SemaphoreType.DMA,
        pltpu.SemaphoreType.DMA,
    ],
)
def _edge_agg(h_hbm, e_hbm, src_hbm, dst_hbm, zeros_hbm, out_hbm,
              idx_v, msg0, msg1, e0, e1, agg_sh,
              si0, si1, se0, se1, sg0, sg1, ss0, ss1):
    cid = lax.axis_index("c")
    sid = lax.axis_index("s")
    wid = sid * SC_CORES + cid
    stripe = pl.ds(sid * ROWS_PER_SUB, ROWS_PER_SUB)
    tail_stripe = pl.ds(SC_SUBCORES * ROWS_PER_SUB, ROWS_TAIL)

    msg = (msg0, msg1)
    ebuf = (e0, e1)
    sem_i = (si0, si1)
    sem_e = (se0, se1)
    sem_g = (sg0, sg1)
    sem_s = (ss0, ss1)

    def kernel_body():
        pltpu.sync_copy(zeros_hbm.at[stripe], agg_sh.at[stripe])

        @pl.when(sid == 0)
        def _():
            pltpu.sync_copy(zeros_hbm.at[tail_stripe], agg_sh.at[tail_stripe])

        plsc.subcore_barrier()

        def start_loads(g, b):
            c = wid + NW * g
            base = c * CHUNK
            pltpu.async_copy(src_hbm.at[pl.ds(base, CHUNK)],
                             idx_v.at[b, 0], sem_i[b])
            pltpu.async_copy(dst_hbm.at[pl.ds(base, CHUNK)],
                             idx_v.at[b, 1], sem_i[b])
            pltpu.async_copy(e_hbm.at[pl.ds(c * HALF, HALF)], ebuf[b],
                             sem_e[b])

        def wait_idx(g, b):
            base = (wid + NW * g) * CHUNK
            pltpu.make_async_copy(src_hbm.at[pl.ds(base, CHUNK)],
                                  idx_v.at[b, 0], sem_i[b]).wait()
            pltpu.make_async_copy(dst_hbm.at[pl.ds(base, CHUNK)],
                                  idx_v.at[b, 1], sem_i[b]).wait()

        def wait_e(b):
            pltpu.make_async_copy(e_hbm.at[pl.ds(0, HALF)],
                                  ebuf[b], sem_e[b]).wait()

        def start_gather(b):
            pltpu.async_copy(h_hbm.at[idx_v.at[b, 0]], msg[b], sem_g[b])

        def wait_gather(b):
            pltpu.make_async_copy(h_hbm.at[idx_v.at[b, 0]],
                                  msg[b], sem_g[b]).wait()

        def start_scatter(b):
            pltpu.async_copy(msg[b], agg_sh.at[idx_v.at[b, 1]], sem_s[b],
                             add=True)

        def wait_scatter(b):
            pltpu.make_async_copy(msg[b], agg_sh.at[idx_v.at[b, 1]],
                                  sem_s[b]).wait()

        def compute(b):
            @plsc.parallel_loop(0, CHUNK // 2, unroll=4)
            def _(p):
                _relu_add_rows(msg[b], ebuf[b], p)

        start_loads(0, 0)
        start_loads(1, 1)
        wait_idx(0, 0)
        start_gather(0)

        def pair_body(kk, carry):
            g = 2 * kk
            wait_idx(g + 1, 1)

            @pl.when(kk > 0)
            def _():
                wait_scatter(1)

            start_gather(1)
            wait_e(0)
            wait_gather(0)
            compute(0)
            start_scatter(0)

            @pl.when(kk < (CH_PER_W // 2) - 1)
            def _():
                start_loads(g + 2, 0)

            @pl.when(kk < (CH_PER_W // 2) - 1)
            def _():
                wait_idx(g + 2, 0)
                wait_scatter(0)
                start_gather(0)

            wait_e(1)
            wait_gather(1)
            compute(1)
            start_scatter(1)

            @pl.when(kk < (CH_PER_W // 2) - 1)
            def _():
                start_loads(g + 3, 1)

            return carry

        lax.fori_loop(0, CH_PER_W // 2, pair_body, 0)
        wait_scatter(0)
        wait_scatter(1)

        @pl.when(wid < LEFTOVER)
        def _():
            c = NW * CH_PER_W + wid
            base = c * CHUNK
            pltpu.sync_copy(src_hbm.at[pl.ds(base, CHUNK)], idx_v.at[0, 0])
            pltpu.sync_copy(dst_hbm.at[pl.ds(base, CHUNK)], idx_v.at[0, 1])
            pltpu.sync_copy(e_hbm.at[pl.ds(c * HALF, HALF)], e0)
            pltpu.async_copy(h_hbm.at[idx_v.at[0, 0]], msg0, sg0).wait()
            compute(0)
            pltpu.sync_copy(msg0, agg_sh.at[idx_v.at[0, 1]], add=True)

        plsc.subcore_barrier()
        pltpu.sync_copy(agg_sh.at[stripe], out_hbm.at[cid, stripe])

        @pl.when(sid == 0)
        def _():
            pltpu.sync_copy(agg_sh.at[tail_stripe],
                            out_hbm.at[cid, tail_stripe])

    kernel_body()


def _mlp_body(eps_ref, h_ref, agg_ref, w1_ref, b1_ref, w2_ref, b2_ref,
              batch_ref, hout_ref, pool_ref):
    eps = eps_ref[0, 0]
    z = h_ref[...] * (1.0 + eps) + agg_ref[0] + agg_ref[1]
    t = jnp.maximum(
        jax.lax.dot_general(z, w1_ref[...], (((1,), (0,)), ((), ())),
                            preferred_element_type=jnp.float32)
        + b1_ref[0][None, :], 0.0)
    o = jnp.maximum(
        jax.lax.dot_general(t, w2_ref[...], (((1,), (0,)), ((), ())),
                            preferred_element_type=jnp.float32)
        + b2_ref[0][None, :], 0.0)
    hout_ref[...] = o
    b = batch_ref[0, 0, :]
    oh = (b[:, None] == lax.broadcasted_iota(jnp.int32, (NODE_BLK, N_GRAPHS), 1)
          ).astype(jnp.float32)
    contrib = jax.lax.dot_general(oh, o, (((0,), (0,)), ((), ())),
                                  preferred_element_type=jnp.float32)

    @pl.when(pl.program_id(0) == 0)
    def _():
        pool_ref[...] = contrib

    @pl.when(pl.program_id(0) != 0)
    def _():
        pool_ref[...] = pool_ref[...] + contrib


def _mlp_layer(eps, h, agg2, w1, b1, w2, b2, batch3):
    return pl.pallas_call(
        _mlp_body,
        grid=(NODE_GRID,),
        in_specs=[
            pl.BlockSpec(memory_space=pltpu.SMEM),
            pl.BlockSpec((NODE_BLK, NHID), lambda i: (i, 0)),
            pl.BlockSpec((SC_CORES, NODE_BLK, NHID), lambda i: (0, i, 0)),
            pl.BlockSpec((NHID, 2 * NHID), lambda i: (0, 0)),
            pl.BlockSpec((1, 2 * NHID), lambda i: (0, 0)),
            pl.BlockSpec((2 * NHID, NHID), lambda i: (0, 0)),
            pl.BlockSpec((1, NHID), lambda i: (0, 0)),
            pl.BlockSpec((1, 1, NODE_BLK), lambda i: (i, 0, 0)),
        ],
        out_specs=[
            pl.BlockSpec((NODE_BLK, NHID), lambda i: (i, 0)),
            pl.BlockSpec((N_GRAPHS, NHID), lambda i: (0, 0)),
        ],
        out_shape=[
            jax.ShapeDtypeStruct((N_NODES, NHID), jnp.float32),
            jax.ShapeDtypeStruct((N_GRAPHS, NHID), jnp.float32),
        ],
    )(eps, h, agg2, w1, b1, w2, b2, batch3)


def kernel(x, edge_index, edge_attr, batch, atom_tables, bond_tables,
           mlp_params):
    atom_tab = jnp.zeros((ATOM_PAD, NHID), jnp.float32)
    atom_tab = atom_tab.at[:ATOM_TOT].set(jnp.concatenate(atom_tables, axis=0))
    bond_tab = jnp.zeros((BOND_PAD, NHID), jnp.float32)
    bond_tab = bond_tab.at[:BOND_TOT].set(jnp.concatenate(bond_tables, axis=0))

    h = _atom_encoder(x, atom_tab)
    e16 = _bond_encoder(edge_attr.T, bond_tab)

    src = edge_index[0]
    dst = edge_index[1]
    zeros = jnp.zeros((N_NODES, NHID), jnp.float32)
    batch3 = batch.reshape(NODE_GRID, 1, NODE_BLK)

    hs = []
    pools = []
    for (w1, b1, w2, b2, eps) in mlp_params:
        agg2 = _edge_agg(h, e16, src, dst, zeros)
        h, pool = _mlp_layer(
            jnp.reshape(eps, (1, 1)), h, agg2, w1,
            jnp.reshape(b1, (1, 2 * NHID)), w2, jnp.reshape(b2, (1, NHID)),
            batch3)
        hs.append(h)
        pools.append(pool)

    node_embs = jnp.concatenate(hs, axis=-1)
    graph_embs = jnp.concatenate(pools, axis=-1)
    return (graph_embs, node_embs)
